# baseline (speedup 1.0000x reference)
import functools
import math

import jax
import jax.numpy as jnp
from jax import lax
from jax.experimental import pallas as pl
from jax.experimental.pallas import tpu as pltpu
from jax.experimental.pallas import tpu_sc as plsc

_NC = 2
_NS = 16
_NW = _NC * _NS
_L = 16

_CF, _RT = 0.25, 128


def _capacity(num_tokens):
    cap = math.ceil(_CF * num_tokens)
    cap = _RT * math.ceil(cap / _RT)
    return max(1, min(cap, num_tokens))


def _router_body(x_ref, wr_ref, eidx_ref, w_ref, cnt_ref, *, tb, e):
    xb = x_ref[...]
    wr = wr_ref[...]
    logits = jnp.dot(xb, wr, preferred_element_type=jnp.float32)
    iota_e = lax.broadcasted_iota(jnp.int32, (tb, e), 1)
    m1 = jnp.max(logits, axis=1, keepdims=True)
    i1 = jnp.min(jnp.where(logits == m1, iota_e, e), axis=1, keepdims=True)
    masked = jnp.where(iota_e == i1, -jnp.inf, logits)
    m2 = jnp.max(masked, axis=1, keepdims=True)
    i2 = jnp.min(jnp.where(masked == m2, iota_e, e), axis=1, keepdims=True)
    w1 = jax.nn.sigmoid(m1 - m2)
    eidx_ref[...] = jnp.concatenate([i1, i2], axis=1)
    w_ref[...] = jnp.concatenate([w1, 1.0 - w1], axis=1)
    oh = (iota_e == i1).astype(jnp.int32) + (iota_e == i2).astype(jnp.int32)
    nch = tb // 128
    for g in range(nch):
        cnt_ref[0, g:g + 1, :] = jnp.sum(
            oh[g * 128:(g + 1) * 128, :], axis=0, keepdims=True)


def _router(x_flat, wr):
    n, h = x_flat.shape
    e = wr.shape[1]
    tb = 512
    grid = (n // tb,)
    return pl.pallas_call(
        functools.partial(_router_body, tb=tb, e=e),
        grid=grid,
        in_specs=[
            pl.BlockSpec((tb, h), lambda i: (i, 0)),
            pl.BlockSpec((h, e), lambda i: (0, 0)),
        ],
        out_specs=[
            pl.BlockSpec((tb, 2), lambda i: (i, 0)),
            pl.BlockSpec((tb, 2), lambda i: (i, 0)),
            pl.BlockSpec((1, tb // 128, e), lambda i: (i, 0, 0)),
        ],
        out_shape=[
            jax.ShapeDtypeStruct((n, 2), jnp.int32),
            jax.ShapeDtypeStruct((n, 2), jnp.float32),
            jax.ShapeDtypeStruct((n // tb, tb // 128, e), jnp.int32),
        ],
    )(x_flat, wr)


def _make_pack(n, h, e, cap):
    a = 2 * n
    ca = a // _NW
    sc = 32
    nsub = ca // sc
    mesh = plsc.VectorSubcoreMesh(core_axis_name="c", subcore_axis_name="s")

    @functools.partial(
        pl.kernel,
        mesh=mesh,
        out_type=[
            jax.ShapeDtypeStruct((e * cap + _NW, h), jnp.float32),
            jax.ShapeDtypeStruct((a,), jnp.int32),
            jax.ShapeDtypeStruct((a,), jnp.float32),
        ],
        scratch_types=[
            pltpu.VMEM((ca,), jnp.int32),
            pltpu.VMEM((ca,), jnp.float32),
            pltpu.VMEM((_NW * e,), jnp.int32),
            pltpu.VMEM((nsub, sc), jnp.int32),
            pltpu.VMEM((nsub, sc), jnp.int32),
            pltpu.VMEM((ca,), jnp.int32),
            pltpu.VMEM((ca,), jnp.float32),
            pltpu.VMEM((sc, h), jnp.float32),
            pltpu.VMEM((sc, h), jnp.float32),
            pltpu.SemaphoreType.DMA,
            pltpu.SemaphoreType.DMA,
            pltpu.SemaphoreType.DMA,
        ],
        compiler_params=pltpu.CompilerParams(needs_layout_passes=False),
    )
    def pack(eidx_hbm, w_hbm, cnt_hbm, x_hbm, xe_hbm, rix_hbm, wk_hbm,
             ev_v, wv_v, ct_v, tok_v, dst_v, rix_v, wk_v, buf_a, buf_b,
             gsem_a, gsem_b, ssem):
        wid = lax.axis_index("s") * _NC + lax.axis_index("c")
        base = wid * ca
        pltpu.sync_copy(eidx_hbm.at[pl.ds(base, ca)], ev_v)
        pltpu.sync_copy(w_hbm.at[pl.ds(base, ca)], wv_v)
        pltpu.sync_copy(cnt_hbm, ct_v)

        iota = lax.iota(jnp.int32, _L)
        carry = []
        for ei in range(e):
            v0 = plsc.load_gather(ct_v, [iota * e + ei])
            v1 = plsc.load_gather(ct_v, [(iota + _L) * e + ei])
            s0 = jnp.sum(jnp.where(iota < wid, v0, 0))
            s1 = jnp.sum(jnp.where(iota + _L < wid, v1, 0))
            carry.append(s0 + s1)

        gpsub = sc // _L
        for g in range(ca // _L):
            ev = ev_v[pl.ds(g * _L, _L)]
            wv = wv_v[pl.ds(g * _L, _L)]
            slot = jnp.zeros((_L,), jnp.int32)
            for ei in range(e):
                ind = ev == ei
                indi = ind.astype(jnp.int32)
                cs = plsc.cumsum(indi)
                slot = slot + jnp.where(ind, cs - indi + carry[ei], 0)
                carry[ei] = carry[ei] + jnp.sum(indi)
            keep = slot < cap
            dest = jnp.where(keep, ev * cap + slot, e * cap + wid)
            rix_v[pl.ds(g * _L, _L)] = dest
            wk_v[pl.ds(g * _L, _L)] = jnp.where(keep, wv, 0.0)
            sub, col = g // gpsub, (g % gpsub) * _L
            dst_v[sub, pl.ds(col, _L)] = dest
            tok_v[sub, pl.ds(col, _L)] = (base + g * _L + iota) >> 1

        pltpu.sync_copy(rix_v, rix_hbm.at[pl.ds(base, ca)])
        pltpu.sync_copy(wk_v, wk_hbm.at[pl.ds(base, ca)])

        bufs = (buf_a, buf_b)
        gsems = (gsem_a, gsem_b)
        gathers = [None] * nsub
        gathers[0] = pltpu.async_copy(x_hbm.at[tok_v.at[0]], bufs[0], gsems[0])
        for sub in range(nsub):
            buf = bufs[sub % 2]
            gathers[sub].wait()
            scat = pltpu.async_copy(buf, xe_hbm.at[dst_v.at[sub]], ssem)
            if sub + 1 < nsub:
                gathers[sub + 1] = pltpu.async_copy(
                    x_hbm.at[tok_v.at[sub + 1]], bufs[(sub + 1) % 2],
                    gsems[(sub + 1) % 2])
            scat.wait()

    return pack


def _mlp_body(x_ref, w1_ref, b1_ref, w2_ref, b2_ref, o_ref, *, ns, fs):
    f = pl.program_id(1)
    x = x_ref[...].astype(jnp.bfloat16)
    contrib = None
    for s in range(ns):
        sl = slice(s * fs, (s + 1) * fs)
        w1 = w1_ref[0][:, sl]
        h = jax.nn.gelu(jnp.dot(x, w1, preferred_element_type=jnp.float32)
                        + b1_ref[0][:, sl])
        c = jnp.dot(h.astype(jnp.bfloat16), w2_ref[0][sl, :],
                    preferred_element_type=jnp.float32)
        contrib = c if contrib is None else contrib + c

    @pl.when(f == 0)
    def _():
        o_ref[...] = contrib + b2_ref[0]

    @pl.when(f != 0)
    def _():
        o_ref[...] += contrib


def _mlp(xe, w1, b1, w2, b2, cap):
    e, h, f = w1.shape
    ft = 1024
    nf = f // ft
    return pl.pallas_call(
        functools.partial(_mlp_body, ns=4, fs=ft // 4),
        grid=(e, nf),
        in_specs=[
            pl.BlockSpec((cap, h), lambda i, j: (i, 0)),
            pl.BlockSpec((1, h, ft), lambda i, j: (i, 0, j)),
            pl.BlockSpec((1, 1, ft), lambda i, j: (i, 0, j)),
            pl.BlockSpec((1, ft, h), lambda i, j: (i, j, 0)),
            pl.BlockSpec((1, 1, h), lambda i, j: (i, 0, 0)),
        ],
        out_specs=pl.BlockSpec((cap, h), lambda i, j: (i, 0)),
        out_shape=jax.ShapeDtypeStruct((e * cap + _NW, h), jnp.float32),
        compiler_params=pltpu.CompilerParams(
            dimension_semantics=("parallel", "arbitrary")),
    )(xe, w1.astype(jnp.bfloat16), b1.reshape(e, 1, f),
      w2.astype(jnp.bfloat16), b2.reshape(e, 1, h))


def _make_combine(n, h, ecap):
    tpw = n // _NW
    tsc = 16
    nsub = tpw // tsc
    mesh = plsc.VectorSubcoreMesh(core_axis_name="c", subcore_axis_name="s")

    @functools.partial(
        pl.kernel,
        mesh=mesh,
        out_type=jax.ShapeDtypeStruct((n, h), jnp.float32),
        scratch_types=[
            pltpu.VMEM((nsub, 2 * tsc), jnp.int32),
            pltpu.VMEM((2 * tpw,), jnp.float32),
            pltpu.VMEM((2 * tsc, h), jnp.float32),
            pltpu.VMEM((2 * tsc, h), jnp.float32),
            pltpu.VMEM((tsc, h), jnp.float32),
            pltpu.VMEM((tsc, h), jnp.float32),
            pltpu.SemaphoreType.DMA,
            pltpu.SemaphoreType.DMA,
            pltpu.SemaphoreType.DMA,
            pltpu.SemaphoreType.DMA,
        ],
        compiler_params=pltpu.CompilerParams(needs_layout_passes=False),
    )
    def combine(y_hbm, rix_hbm, wk_hbm, out_hbm, rix_v, wk_v, ybuf_a, ybuf_b,
                obuf_a, obuf_b, sem_a, sem_b, osem_a, osem_b):
        wid = lax.axis_index("s") * _NC + lax.axis_index("c")
        tbase = wid * tpw
        abase = 2 * tbase
        zeros = jnp.zeros((_L,), jnp.float32)
        for v in range(h // _L):
            obuf_a[0, pl.ds(v * _L, _L)] = zeros
        pltpu.sync_copy(obuf_a.at[pl.ds(0, 1)],
                        y_hbm.at[pl.ds(ecap + wid, 1)])
        for sub in range(nsub):
            pltpu.sync_copy(rix_hbm.at[pl.ds(abase + sub * 2 * tsc, 2 * tsc)],
                            rix_v.at[sub])
        pltpu.sync_copy(wk_hbm.at[pl.ds(abase, 2 * tpw)], wk_v)

        bufs = (ybuf_a, ybuf_b)
        sems = (sem_a, sem_b)
        obufs = (obuf_a, obuf_b)
        osems = (osem_a, osem_b)
        gathers = [None] * nsub
        owrites = [None] * nsub
        gathers[0] = pltpu.async_copy(y_hbm.at[rix_v.at[0]], bufs[0], sems[0])
        for sub in range(nsub):
            ybuf = bufs[sub % 2]
            obuf = obufs[sub % 2]
            gathers[sub].wait()
            if sub + 1 < nsub:
                gathers[sub + 1] = pltpu.async_copy(
                    y_hbm.at[rix_v.at[sub + 1]], bufs[(sub + 1) % 2],
                    sems[(sub + 1) % 2])
            if sub >= 2:
                owrites[sub - 2].wait()

            def body(i, _, sub=sub, ybuf=ybuf, obuf=obuf):
                j0 = jnp.broadcast_to(sub * 2 * tsc + 4 * i,
                                      (_L,)).astype(jnp.int32)
                for p in range(2):
                    w0 = plsc.load_gather(wk_v, [j0 + 2 * p])
                    w1 = plsc.load_gather(wk_v, [j0 + 2 * p + 1])
                    t = 2 * i + p
                    for v in range(h // _L):
                        ya = ybuf[2 * t, pl.ds(v * _L, _L)]
                        yb = ybuf[2 * t + 1, pl.ds(v * _L, _L)]
                        obuf[t, pl.ds(v * _L, _L)] = ya * w0 + yb * w1
                return 0

            lax.fori_loop(0, tsc // 2, body, 0)
            owrites[sub] = pltpu.async_copy(
                obuf, out_hbm.at[pl.ds(tbase + sub * tsc, tsc)],
                osems[sub % 2])
        owrites[nsub - 2].wait()
        owrites[nsub - 1].wait()

    return combine


def kernel(x, Wr, W1, b1, W2, b2):
    bx, tx, hx = x.shape
    n = bx * tx
    e = Wr.shape[1]
    cap = _capacity(n)
    x_flat = x.reshape(n, hx)

    eidx, w, counts = _router(x_flat, Wr)
    pack = _make_pack(n, hx, e, cap)
    xe, rix, wk = pack(eidx.reshape(-1), w.reshape(-1),
                       counts.reshape(-1), x_flat)
    y = _mlp(xe, W1, b1, W2, b2, cap)
    combine = _make_combine(n, hx, e * cap)
    out = combine(y, rix, wk)
    return out.reshape(bx, tx, hx)


# speedup vs baseline: 1.3230x; 1.3230x over previous
import functools
import math

import jax
import jax.numpy as jnp
from jax import lax
from jax.experimental import pallas as pl
from jax.experimental.pallas import tpu as pltpu
from jax.experimental.pallas import tpu_sc as plsc

_NC = 2
_NS = 16
_NW = _NC * _NS
_L = 16

_CF, _RT = 0.25, 128


def _capacity(num_tokens):
    cap = math.ceil(_CF * num_tokens)
    cap = _RT * math.ceil(cap / _RT)
    return max(1, min(cap, num_tokens))


def _router_body(x_ref, wr_ref, eidx_ref, w_ref, cnt_ref, *, tb, e):
    xb = x_ref[...]
    wr = wr_ref[...]
    logits = jnp.dot(xb, wr, preferred_element_type=jnp.float32)
    iota_e = lax.broadcasted_iota(jnp.int32, (tb, e), 1)
    m1 = jnp.max(logits, axis=1, keepdims=True)
    i1 = jnp.min(jnp.where(logits == m1, iota_e, e), axis=1, keepdims=True)
    masked = jnp.where(iota_e == i1, -jnp.inf, logits)
    m2 = jnp.max(masked, axis=1, keepdims=True)
    i2 = jnp.min(jnp.where(masked == m2, iota_e, e), axis=1, keepdims=True)
    w1 = jax.nn.sigmoid(m1 - m2)
    eidx_ref[...] = jnp.concatenate([i1, i2], axis=1)
    w_ref[...] = jnp.concatenate([w1, 1.0 - w1], axis=1)
    oh = (iota_e == i1).astype(jnp.int32) + (iota_e == i2).astype(jnp.int32)
    nch = tb // 128
    for g in range(nch):
        cnt_ref[0, g:g + 1, :] = jnp.sum(
            oh[g * 128:(g + 1) * 128, :], axis=0, keepdims=True)


def _router(x_flat, wr):
    n, h = x_flat.shape
    e = wr.shape[1]
    tb = 512
    grid = (n // tb,)
    return pl.pallas_call(
        functools.partial(_router_body, tb=tb, e=e),
        grid=grid,
        in_specs=[
            pl.BlockSpec((tb, h), lambda i: (i, 0)),
            pl.BlockSpec((h, e), lambda i: (0, 0)),
        ],
        out_specs=[
            pl.BlockSpec((tb, 2), lambda i: (i, 0)),
            pl.BlockSpec((tb, 2), lambda i: (i, 0)),
            pl.BlockSpec((1, tb // 128, e), lambda i: (i, 0, 0)),
        ],
        out_shape=[
            jax.ShapeDtypeStruct((n, 2), jnp.int32),
            jax.ShapeDtypeStruct((n, 2), jnp.float32),
            jax.ShapeDtypeStruct((n // tb, tb // 128, e), jnp.int32),
        ],
    )(x_flat, wr)


def _make_pack(n, h, e, cap):
    a = 2 * n
    ca = a // _NW
    sc = 32
    nsub = ca // sc
    mesh = plsc.VectorSubcoreMesh(core_axis_name="c", subcore_axis_name="s")

    @functools.partial(
        pl.kernel,
        mesh=mesh,
        out_type=[
            jax.ShapeDtypeStruct((e * cap + _NW, h), jnp.float32),
            jax.ShapeDtypeStruct((a,), jnp.int32),
            jax.ShapeDtypeStruct((a,), jnp.float32),
        ],
        scratch_types=[
            pltpu.VMEM((ca,), jnp.int32),
            pltpu.VMEM((ca,), jnp.float32),
            pltpu.VMEM((_NW * e,), jnp.int32),
            pltpu.VMEM((nsub, sc), jnp.int32),
            pltpu.VMEM((nsub, sc), jnp.int32),
            pltpu.VMEM((ca,), jnp.int32),
            pltpu.VMEM((ca,), jnp.float32),
            pltpu.VMEM((sc, h), jnp.float32),
            pltpu.VMEM((sc, h), jnp.float32),
            pltpu.SemaphoreType.DMA,
            pltpu.SemaphoreType.DMA,
            pltpu.SemaphoreType.DMA,
        ],
        compiler_params=pltpu.CompilerParams(needs_layout_passes=False),
    )
    def pack(eidx_hbm, w_hbm, cnt_hbm, x_hbm, xe_hbm, rix_hbm, wk_hbm,
             ev_v, wv_v, ct_v, tok_v, dst_v, rix_v, wk_v, buf_a, buf_b,
             gsem_a, gsem_b, ssem):
        wid = lax.axis_index("s") * _NC + lax.axis_index("c")
        base = wid * ca
        pltpu.sync_copy(eidx_hbm.at[pl.ds(base, ca)], ev_v)
        pltpu.sync_copy(w_hbm.at[pl.ds(base, ca)], wv_v)
        pltpu.sync_copy(cnt_hbm, ct_v)

        iota = lax.iota(jnp.int32, _L)
        carry = []
        for ei in range(e):
            v0 = plsc.load_gather(ct_v, [iota * e + ei])
            v1 = plsc.load_gather(ct_v, [(iota + _L) * e + ei])
            s0 = jnp.sum(jnp.where(iota < wid, v0, 0))
            s1 = jnp.sum(jnp.where(iota + _L < wid, v1, 0))
            carry.append(s0 + s1)

        gpsub = sc // _L
        for g in range(ca // _L):
            ev = ev_v[pl.ds(g * _L, _L)]
            wv = wv_v[pl.ds(g * _L, _L)]
            slot = jnp.zeros((_L,), jnp.int32)
            for ei in range(e):
                ind = ev == ei
                indi = ind.astype(jnp.int32)
                cs = plsc.cumsum(indi)
                slot = slot + jnp.where(ind, cs - indi + carry[ei], 0)
                carry[ei] = carry[ei] + jnp.sum(indi)
            keep = slot < cap
            dest = jnp.where(keep, ev * cap + slot, e * cap + wid)
            rix_v[pl.ds(g * _L, _L)] = dest
            wk_v[pl.ds(g * _L, _L)] = jnp.where(keep, wv, 0.0)
            sub, col = g // gpsub, (g % gpsub) * _L
            dst_v[sub, pl.ds(col, _L)] = dest
            tok_v[sub, pl.ds(col, _L)] = (base + g * _L + iota) >> 1

        pltpu.sync_copy(rix_v, rix_hbm.at[pl.ds(base, ca)])
        pltpu.sync_copy(wk_v, wk_hbm.at[pl.ds(base, ca)])

        bufs = (buf_a, buf_b)
        gsems = (gsem_a, gsem_b)
        gathers = [None] * nsub
        gathers[0] = pltpu.async_copy(x_hbm.at[tok_v.at[0]], bufs[0], gsems[0])
        for sub in range(nsub):
            buf = bufs[sub % 2]
            gathers[sub].wait()
            scat = pltpu.async_copy(buf, xe_hbm.at[dst_v.at[sub]], ssem)
            if sub + 1 < nsub:
                gathers[sub + 1] = pltpu.async_copy(
                    x_hbm.at[tok_v.at[sub + 1]], bufs[(sub + 1) % 2],
                    gsems[(sub + 1) % 2])
            scat.wait()

    return pack


def _mlp_body(x_ref, w1_ref, b1_ref, w2_ref, b2_ref, o_ref, *, ns, fs):
    f = pl.program_id(1)
    x = x_ref[...].astype(jnp.bfloat16)
    contrib = None
    for s in range(ns):
        sl = slice(s * fs, (s + 1) * fs)
        w1 = w1_ref[0][:, sl].astype(jnp.bfloat16)
        h = jax.nn.gelu(jnp.dot(x, w1, preferred_element_type=jnp.float32)
                        + b1_ref[0][:, sl])
        c = jnp.dot(h.astype(jnp.bfloat16), w2_ref[0][sl, :].astype(jnp.bfloat16),
                    preferred_element_type=jnp.float32)
        contrib = c if contrib is None else contrib + c

    @pl.when(f == 0)
    def _():
        o_ref[...] = contrib + b2_ref[0]

    @pl.when(f != 0)
    def _():
        o_ref[...] += contrib


def _mlp(xe, w1, b1, w2, b2, cap):
    e, h, f = w1.shape
    ft = 1024
    nf = f // ft
    return pl.pallas_call(
        functools.partial(_mlp_body, ns=4, fs=ft // 4),
        grid=(e, nf),
        in_specs=[
            pl.BlockSpec((cap, h), lambda i, j: (i, 0)),
            pl.BlockSpec((1, h, ft), lambda i, j: (i, 0, j)),
            pl.BlockSpec((1, 1, ft), lambda i, j: (i, 0, j)),
            pl.BlockSpec((1, ft, h), lambda i, j: (i, j, 0)),
            pl.BlockSpec((1, 1, h), lambda i, j: (i, 0, 0)),
        ],
        out_specs=pl.BlockSpec((cap, h), lambda i, j: (i, 0)),
        out_shape=jax.ShapeDtypeStruct((e * cap + _NW, h), jnp.float32),
        compiler_params=pltpu.CompilerParams(
            dimension_semantics=("parallel", "arbitrary")),
    )(xe, w1, b1.reshape(e, 1, f), w2, b2.reshape(e, 1, h))


def _make_combine(n, h, ecap):
    tpw = n // _NW
    tsc = 16
    nsub = tpw // tsc
    mesh = plsc.VectorSubcoreMesh(core_axis_name="c", subcore_axis_name="s")

    @functools.partial(
        pl.kernel,
        mesh=mesh,
        out_type=jax.ShapeDtypeStruct((n, h), jnp.float32),
        scratch_types=[
            pltpu.VMEM((nsub, 2 * tsc), jnp.int32),
            pltpu.VMEM((2 * tpw,), jnp.float32),
            pltpu.VMEM((2 * tsc, h), jnp.float32),
            pltpu.VMEM((2 * tsc, h), jnp.float32),
            pltpu.VMEM((tsc, h), jnp.float32),
            pltpu.VMEM((tsc, h), jnp.float32),
            pltpu.SemaphoreType.DMA,
            pltpu.SemaphoreType.DMA,
            pltpu.SemaphoreType.DMA,
            pltpu.SemaphoreType.DMA,
        ],
        compiler_params=pltpu.CompilerParams(needs_layout_passes=False),
    )
    def combine(y_hbm, rix_hbm, wk_hbm, out_hbm, rix_v, wk_v, ybuf_a, ybuf_b,
                obuf_a, obuf_b, sem_a, sem_b, osem_a, osem_b):
        wid = lax.axis_index("s") * _NC + lax.axis_index("c")
        tbase = wid * tpw
        abase = 2 * tbase
        zeros = jnp.zeros((_L,), jnp.float32)
        for v in range(h // _L):
            obuf_a[0, pl.ds(v * _L, _L)] = zeros
        pltpu.sync_copy(obuf_a.at[pl.ds(0, 1)],
                        y_hbm.at[pl.ds(ecap + wid, 1)])
        for sub in range(nsub):
            pltpu.sync_copy(rix_hbm.at[pl.ds(abase + sub * 2 * tsc, 2 * tsc)],
                            rix_v.at[sub])
        pltpu.sync_copy(wk_hbm.at[pl.ds(abase, 2 * tpw)], wk_v)

        bufs = (ybuf_a, ybuf_b)
        sems = (sem_a, sem_b)
        obufs = (obuf_a, obuf_b)
        osems = (osem_a, osem_b)
        gathers = [None] * nsub
        owrites = [None] * nsub
        gathers[0] = pltpu.async_copy(y_hbm.at[rix_v.at[0]], bufs[0], sems[0])
        for sub in range(nsub):
            ybuf = bufs[sub % 2]
            obuf = obufs[sub % 2]
            gathers[sub].wait()
            if sub + 1 < nsub:
                gathers[sub + 1] = pltpu.async_copy(
                    y_hbm.at[rix_v.at[sub + 1]], bufs[(sub + 1) % 2],
                    sems[(sub + 1) % 2])
            if sub >= 2:
                owrites[sub - 2].wait()

            def body(i, _, sub=sub, ybuf=ybuf, obuf=obuf):
                j0 = jnp.broadcast_to(sub * 2 * tsc + 4 * i,
                                      (_L,)).astype(jnp.int32)
                for p in range(2):
                    w0 = plsc.load_gather(wk_v, [j0 + 2 * p])
                    w1 = plsc.load_gather(wk_v, [j0 + 2 * p + 1])
                    t = 2 * i + p
                    for v in range(h // _L):
                        ya = ybuf[2 * t, pl.ds(v * _L, _L)]
                        yb = ybuf[2 * t + 1, pl.ds(v * _L, _L)]
                        obuf[t, pl.ds(v * _L, _L)] = ya * w0 + yb * w1
                return 0

            lax.fori_loop(0, tsc // 2, body, 0)
            owrites[sub] = pltpu.async_copy(
                obuf, out_hbm.at[pl.ds(tbase + sub * tsc, tsc)],
                osems[sub % 2])
        owrites[nsub - 2].wait()
        owrites[nsub - 1].wait()

    return combine


def kernel(x, Wr, W1, b1, W2, b2):
    bx, tx, hx = x.shape
    n = bx * tx
    e = Wr.shape[1]
    cap = _capacity(n)
    x_flat = x.reshape(n, hx)

    eidx, w, counts = _router(x_flat, Wr)
    pack = _make_pack(n, hx, e, cap)
    xe, rix, wk = pack(eidx.reshape(-1), w.reshape(-1),
                       counts.reshape(-1), x_flat)
    y = _mlp(xe, W1, b1, W2, b2, cap)
    combine = _make_combine(n, hx, e * cap)
    out = combine(y, rix, wk)
    return out.reshape(bx, tx, hx)


# speedup vs baseline: 1.3341x; 1.0084x over previous
import functools
import math

import jax
import jax.numpy as jnp
from jax import lax
from jax.experimental import pallas as pl
from jax.experimental.pallas import tpu as pltpu
from jax.experimental.pallas import tpu_sc as plsc

_NC = 2
_NS = 16
_NW = _NC * _NS
_L = 16

_CF, _RT = 0.25, 128


def _capacity(num_tokens):
    cap = math.ceil(_CF * num_tokens)
    cap = _RT * math.ceil(cap / _RT)
    return max(1, min(cap, num_tokens))


def _router_body(x_ref, wr_ref, eidx_ref, w_ref, cnt_ref, *, tb, e):
    xb = x_ref[...]
    wr = wr_ref[...]
    logits = jnp.dot(xb, wr, preferred_element_type=jnp.float32)
    iota_e = lax.broadcasted_iota(jnp.int32, (tb, e), 1)
    m1 = jnp.max(logits, axis=1, keepdims=True)
    i1 = jnp.min(jnp.where(logits == m1, iota_e, e), axis=1, keepdims=True)
    masked = jnp.where(iota_e == i1, -jnp.inf, logits)
    m2 = jnp.max(masked, axis=1, keepdims=True)
    i2 = jnp.min(jnp.where(masked == m2, iota_e, e), axis=1, keepdims=True)
    w1 = jax.nn.sigmoid(m1 - m2)
    eidx_ref[...] = jnp.concatenate([i1, i2], axis=1)
    w_ref[...] = jnp.concatenate([w1, 1.0 - w1], axis=1)
    oh = (iota_e == i1).astype(jnp.int32) + (iota_e == i2).astype(jnp.int32)
    nch = tb // 128
    for g in range(nch):
        cnt_ref[0, g:g + 1, :] = jnp.sum(
            oh[g * 128:(g + 1) * 128, :], axis=0, keepdims=True)


def _router(x_flat, wr):
    n, h = x_flat.shape
    e = wr.shape[1]
    tb = 512
    grid = (n // tb,)
    return pl.pallas_call(
        functools.partial(_router_body, tb=tb, e=e),
        grid=grid,
        in_specs=[
            pl.BlockSpec((tb, h), lambda i: (i, 0)),
            pl.BlockSpec((h, e), lambda i: (0, 0)),
        ],
        out_specs=[
            pl.BlockSpec((tb, 2), lambda i: (i, 0)),
            pl.BlockSpec((tb, 2), lambda i: (i, 0)),
            pl.BlockSpec((1, tb // 128, e), lambda i: (i, 0, 0)),
        ],
        out_shape=[
            jax.ShapeDtypeStruct((n, 2), jnp.int32),
            jax.ShapeDtypeStruct((n, 2), jnp.float32),
            jax.ShapeDtypeStruct((n // tb, tb // 128, e), jnp.int32),
        ],
    )(x_flat, wr)


def _make_pack(n, h, e, cap):
    a = 2 * n
    ca = a // _NW
    sc = 32
    nsub = ca // sc
    mesh = plsc.VectorSubcoreMesh(core_axis_name="c", subcore_axis_name="s")

    @functools.partial(
        pl.kernel,
        mesh=mesh,
        out_type=[
            jax.ShapeDtypeStruct((e * cap + _NW, h), jnp.float32),
            jax.ShapeDtypeStruct((a,), jnp.int32),
            jax.ShapeDtypeStruct((a,), jnp.float32),
        ],
        scratch_types=[
            pltpu.VMEM((ca,), jnp.int32),
            pltpu.VMEM((ca,), jnp.float32),
            pltpu.VMEM((_NW * e,), jnp.int32),
            pltpu.VMEM((nsub, sc), jnp.int32),
            pltpu.VMEM((nsub, sc), jnp.int32),
            pltpu.VMEM((ca,), jnp.int32),
            pltpu.VMEM((ca,), jnp.float32),
            pltpu.VMEM((sc, h), jnp.float32),
            pltpu.VMEM((sc, h), jnp.float32),
            pltpu.SemaphoreType.DMA,
            pltpu.SemaphoreType.DMA,
            pltpu.SemaphoreType.DMA,
        ],
        compiler_params=pltpu.CompilerParams(needs_layout_passes=False),
    )
    def pack(eidx_hbm, w_hbm, cnt_hbm, x_hbm, xe_hbm, rix_hbm, wk_hbm,
             ev_v, wv_v, ct_v, tok_v, dst_v, rix_v, wk_v, buf_a, buf_b,
             gsem_a, gsem_b, ssem):
        wid = lax.axis_index("s") * _NC + lax.axis_index("c")
        base = wid * ca
        pltpu.sync_copy(eidx_hbm.at[pl.ds(base, ca)], ev_v)
        pltpu.sync_copy(w_hbm.at[pl.ds(base, ca)], wv_v)
        pltpu.sync_copy(cnt_hbm, ct_v)

        iota = lax.iota(jnp.int32, _L)
        carry = []
        for ei in range(e):
            v0 = plsc.load_gather(ct_v, [iota * e + ei])
            v1 = plsc.load_gather(ct_v, [(iota + _L) * e + ei])
            s0 = jnp.sum(jnp.where(iota < wid, v0, 0))
            s1 = jnp.sum(jnp.where(iota + _L < wid, v1, 0))
            carry.append(s0 + s1)

        gpsub = sc // _L
        for g in range(ca // _L):
            ev = ev_v[pl.ds(g * _L, _L)]
            wv = wv_v[pl.ds(g * _L, _L)]
            slot = jnp.zeros((_L,), jnp.int32)
            for ei in range(e):
                ind = ev == ei
                indi = ind.astype(jnp.int32)
                cs = plsc.cumsum(indi)
                slot = slot + jnp.where(ind, cs - indi + carry[ei], 0)
                carry[ei] = carry[ei] + jnp.sum(indi)
            keep = slot < cap
            dest = jnp.where(keep, ev * cap + slot, e * cap + wid)
            rix_v[pl.ds(g * _L, _L)] = dest
            wk_v[pl.ds(g * _L, _L)] = jnp.where(keep, wv, 0.0)
            sub, col = g // gpsub, (g % gpsub) * _L
            dst_v[sub, pl.ds(col, _L)] = dest
            tok_v[sub, pl.ds(col, _L)] = (base + g * _L + iota) >> 1

        pltpu.sync_copy(rix_v, rix_hbm.at[pl.ds(base, ca)])
        pltpu.sync_copy(wk_v, wk_hbm.at[pl.ds(base, ca)])

        bufs = (buf_a, buf_b)
        gsems = (gsem_a, gsem_b)
        gathers = [None] * nsub
        gathers[0] = pltpu.async_copy(x_hbm.at[tok_v.at[0]], bufs[0], gsems[0])
        for sub in range(nsub):
            buf = bufs[sub % 2]
            gathers[sub].wait()
            scat = pltpu.async_copy(buf, xe_hbm.at[dst_v.at[sub]], ssem)
            if sub + 1 < nsub:
                gathers[sub + 1] = pltpu.async_copy(
                    x_hbm.at[tok_v.at[sub + 1]], bufs[(sub + 1) % 2],
                    gsems[(sub + 1) % 2])
            scat.wait()

    return pack


def _mlp_body(x_ref, w1_ref, b1_ref, w2_ref, b2_ref, o_ref, *, ns, fs):
    f = pl.program_id(1)
    x = x_ref[...].astype(jnp.bfloat16)
    contrib = None
    for s in range(ns):
        sl = slice(s * fs, (s + 1) * fs)
        w1 = w1_ref[0][:, sl].astype(jnp.bfloat16)
        h = jax.nn.gelu(jnp.dot(x, w1, preferred_element_type=jnp.float32)
                        + b1_ref[0][:, sl])
        c = jnp.dot(h.astype(jnp.bfloat16), w2_ref[0][sl, :].astype(jnp.bfloat16),
                    preferred_element_type=jnp.float32)
        contrib = c if contrib is None else contrib + c

    @pl.when(f == 0)
    def _():
        o_ref[...] = contrib + b2_ref[0]

    @pl.when(f != 0)
    def _():
        o_ref[...] += contrib


def _mlp(xe, w1, b1, w2, b2, cap):
    e, h, f = w1.shape
    ft = 1024
    nf = f // ft
    return pl.pallas_call(
        functools.partial(_mlp_body, ns=4, fs=ft // 4),
        grid=(e, nf),
        in_specs=[
            pl.BlockSpec((cap, h), lambda i, j: (i, 0)),
            pl.BlockSpec((1, h, ft), lambda i, j: (i, 0, j)),
            pl.BlockSpec((1, 1, ft), lambda i, j: (i, 0, j)),
            pl.BlockSpec((1, ft, h), lambda i, j: (i, j, 0)),
            pl.BlockSpec((1, 1, h), lambda i, j: (i, 0, 0)),
        ],
        out_specs=pl.BlockSpec((cap, h), lambda i, j: (i, 0)),
        out_shape=jax.ShapeDtypeStruct((e * cap + _NW, h), jnp.float32),
        compiler_params=pltpu.CompilerParams(
            dimension_semantics=("parallel", "arbitrary")),
    )(xe, w1, b1.reshape(e, 1, f), w2, b2.reshape(e, 1, h))


def _make_combine(n, h, ecap):
    tpw = n // _NW
    tsc = 16
    nsub = tpw // tsc
    mesh = plsc.VectorSubcoreMesh(core_axis_name="c", subcore_axis_name="s")

    @functools.partial(
        pl.kernel,
        mesh=mesh,
        out_type=jax.ShapeDtypeStruct((n, h), jnp.float32),
        scratch_types=[
            pltpu.VMEM((2 * tpw,), jnp.int32),
            pltpu.VMEM((2 * tpw,), jnp.float32),
            pltpu.VMEM((2 * tsc, h), jnp.float32),
            pltpu.VMEM((2 * tsc, h), jnp.float32),
            pltpu.VMEM((tsc, h), jnp.float32),
            pltpu.VMEM((tsc, h), jnp.float32),
            pltpu.SemaphoreType.DMA,
            pltpu.SemaphoreType.DMA,
            pltpu.SemaphoreType.DMA,
            pltpu.SemaphoreType.DMA,
        ],
        compiler_params=pltpu.CompilerParams(needs_layout_passes=False),
    )
    def combine(y_hbm, rix_hbm, wk_hbm, out_hbm, rix_v, wk_v, ybuf_a, ybuf_b,
                obuf_a, obuf_b, sem_a, sem_b, osem_a, osem_b):
        wid = lax.axis_index("s") * _NC + lax.axis_index("c")
        tbase = wid * tpw
        abase = 2 * tbase
        zeros = jnp.zeros((_L,), jnp.float32)
        for v in range(h // _L):
            obuf_a[0, pl.ds(v * _L, _L)] = zeros
        pltpu.sync_copy(obuf_a.at[pl.ds(0, 1)],
                        y_hbm.at[pl.ds(ecap + wid, 1)])
        pltpu.sync_copy(rix_hbm.at[pl.ds(abase, 2 * tpw)], rix_v)
        pltpu.sync_copy(wk_hbm.at[pl.ds(abase, 2 * tpw)], wk_v)

        bufs = (ybuf_a, ybuf_b)
        sems = (sem_a, sem_b)
        obufs = (obuf_a, obuf_b)
        osems = (osem_a, osem_b)
        gathers = [None] * nsub
        owrites = [None] * nsub
        def g_idx(sub):
            return rix_v.at[pl.ds(sub * 2 * tsc, 2 * tsc)]

        gathers[0] = pltpu.async_copy(y_hbm.at[g_idx(0)], bufs[0], sems[0])
        for sub in range(nsub):
            ybuf = bufs[sub % 2]
            obuf = obufs[sub % 2]
            gathers[sub].wait()
            if sub + 1 < nsub:
                gathers[sub + 1] = pltpu.async_copy(
                    y_hbm.at[g_idx(sub + 1)], bufs[(sub + 1) % 2],
                    sems[(sub + 1) % 2])
            if sub >= 2:
                owrites[sub - 2].wait()

            def body(i, _, sub=sub, ybuf=ybuf, obuf=obuf):
                j0 = jnp.broadcast_to(sub * 2 * tsc + 4 * i,
                                      (_L,)).astype(jnp.int32)
                for p in range(2):
                    w0 = plsc.load_gather(wk_v, [j0 + 2 * p])
                    w1 = plsc.load_gather(wk_v, [j0 + 2 * p + 1])
                    t = 2 * i + p
                    for v in range(h // _L):
                        ya = ybuf[2 * t, pl.ds(v * _L, _L)]
                        yb = ybuf[2 * t + 1, pl.ds(v * _L, _L)]
                        obuf[t, pl.ds(v * _L, _L)] = ya * w0 + yb * w1
                return 0

            lax.fori_loop(0, tsc // 2, body, 0)
            owrites[sub] = pltpu.async_copy(
                obuf, out_hbm.at[pl.ds(tbase + sub * tsc, tsc)],
                osems[sub % 2])
        owrites[nsub - 2].wait()
        owrites[nsub - 1].wait()

    return combine


def kernel(x, Wr, W1, b1, W2, b2):
    bx, tx, hx = x.shape
    n = bx * tx
    e = Wr.shape[1]
    cap = _capacity(n)
    x_flat = x.reshape(n, hx)

    eidx, w, counts = _router(x_flat, Wr)
    pack = _make_pack(n, hx, e, cap)
    xe, rix, wk = pack(eidx.reshape(-1), w.reshape(-1),
                       counts.reshape(-1), x_flat)
    y = _mlp(xe, W1, b1, W2, b2, cap)
    combine = _make_combine(n, hx, e * cap)
    out = combine(y, rix, wk)
    return out.reshape(bx, tx, hx)


# speedup vs baseline: 1.4532x; 1.0893x over previous
import functools
import math

import jax
import jax.numpy as jnp
from jax import lax
from jax.experimental import pallas as pl
from jax.experimental.pallas import tpu as pltpu
from jax.experimental.pallas import tpu_sc as plsc

_NC = 2
_NS = 16
_NW = _NC * _NS
_L = 16

_CF, _RT = 0.25, 128


def _capacity(num_tokens):
    cap = math.ceil(_CF * num_tokens)
    cap = _RT * math.ceil(cap / _RT)
    return max(1, min(cap, num_tokens))


def _router_body(x_ref, wr_ref, eidx_ref, w_ref, cnt_ref, *, tb, e):
    xb = x_ref[...]
    wr = wr_ref[...]
    logits = jnp.dot(xb, wr, preferred_element_type=jnp.float32)
    iota_e = lax.broadcasted_iota(jnp.int32, (tb, e), 1)
    m1 = jnp.max(logits, axis=1, keepdims=True)
    i1 = jnp.min(jnp.where(logits == m1, iota_e, e), axis=1, keepdims=True)
    masked = jnp.where(iota_e == i1, -jnp.inf, logits)
    m2 = jnp.max(masked, axis=1, keepdims=True)
    i2 = jnp.min(jnp.where(masked == m2, iota_e, e), axis=1, keepdims=True)
    w1 = jax.nn.sigmoid(m1 - m2)
    eidx_ref[...] = jnp.concatenate([i1, i2], axis=1)
    w_ref[...] = jnp.concatenate([w1, 1.0 - w1], axis=1)
    oh = (iota_e == i1).astype(jnp.int32) + (iota_e == i2).astype(jnp.int32)
    nch = tb // 128
    for g in range(nch):
        cnt_ref[0, g:g + 1, :] = jnp.sum(
            oh[g * 128:(g + 1) * 128, :], axis=0, keepdims=True)


def _router(x_flat, wr):
    n, h = x_flat.shape
    e = wr.shape[1]
    tb = 512
    grid = (n // tb,)
    return pl.pallas_call(
        functools.partial(_router_body, tb=tb, e=e),
        grid=grid,
        in_specs=[
            pl.BlockSpec((tb, h), lambda i: (i, 0)),
            pl.BlockSpec((h, e), lambda i: (0, 0)),
        ],
        out_specs=[
            pl.BlockSpec((tb, 2), lambda i: (i, 0)),
            pl.BlockSpec((tb, 2), lambda i: (i, 0)),
            pl.BlockSpec((1, tb // 128, e), lambda i: (i, 0, 0)),
        ],
        out_shape=[
            jax.ShapeDtypeStruct((n, 2), jnp.int32),
            jax.ShapeDtypeStruct((n, 2), jnp.float32),
            jax.ShapeDtypeStruct((n // tb, tb // 128, e), jnp.int32),
        ],
    )(x_flat, wr)


def _make_pack(n, h, e, cap):
    a = 2 * n
    ca = a // _NW
    sc = 32
    nsub = ca // sc
    mesh = plsc.VectorSubcoreMesh(core_axis_name="c", subcore_axis_name="s")

    @functools.partial(
        pl.kernel,
        mesh=mesh,
        out_type=[
            jax.ShapeDtypeStruct((e * cap + _NW, h), jnp.float32),
            jax.ShapeDtypeStruct((a,), jnp.int32),
            jax.ShapeDtypeStruct((a,), jnp.float32),
        ],
        scratch_types=[
            pltpu.VMEM((ca,), jnp.int32),
            pltpu.VMEM((ca,), jnp.float32),
            pltpu.VMEM((_NW * e,), jnp.int32),
            pltpu.VMEM((nsub, sc), jnp.int32),
            pltpu.VMEM((nsub, sc), jnp.int32),
            pltpu.VMEM((ca,), jnp.int32),
            pltpu.VMEM((ca,), jnp.float32),
            pltpu.VMEM((sc, h), jnp.float32),
            pltpu.VMEM((sc, h), jnp.float32),
            pltpu.SemaphoreType.DMA,
            pltpu.SemaphoreType.DMA,
            pltpu.SemaphoreType.DMA,
        ],
        compiler_params=pltpu.CompilerParams(needs_layout_passes=False),
    )
    def pack(eidx_hbm, w_hbm, cnt_hbm, x_hbm, xe_hbm, rix_hbm, wk_hbm,
             ev_v, wv_v, ct_v, tok_v, dst_v, rix_v, wk_v, buf_a, buf_b,
             gsem_a, gsem_b, ssem):
        wid = lax.axis_index("s") * _NC + lax.axis_index("c")
        base = wid * ca
        pltpu.sync_copy(eidx_hbm.at[pl.ds(base, ca)], ev_v)
        pltpu.sync_copy(w_hbm.at[pl.ds(base, ca)], wv_v)
        pltpu.sync_copy(cnt_hbm, ct_v)

        iota = lax.iota(jnp.int32, _L)
        carry = []
        for ei in range(e):
            v0 = plsc.load_gather(ct_v, [iota * e + ei])
            v1 = plsc.load_gather(ct_v, [(iota + _L) * e + ei])
            s0 = jnp.sum(jnp.where(iota < wid, v0, 0))
            s1 = jnp.sum(jnp.where(iota + _L < wid, v1, 0))
            carry.append(s0 + s1)

        gpsub = sc // _L
        for g in range(ca // _L):
            ev = ev_v[pl.ds(g * _L, _L)]
            wv = wv_v[pl.ds(g * _L, _L)]
            slot = jnp.zeros((_L,), jnp.int32)
            for ei in range(e):
                ind = ev == ei
                indi = ind.astype(jnp.int32)
                cs = plsc.cumsum(indi)
                slot = slot + jnp.where(ind, cs - indi + carry[ei], 0)
                carry[ei] = carry[ei] + jnp.sum(indi)
            keep = slot < cap
            dest = jnp.where(keep, ev * cap + slot, e * cap + wid)
            rix_v[pl.ds(g * _L, _L)] = dest
            wk_v[pl.ds(g * _L, _L)] = jnp.where(keep, wv, 0.0)
            sub, col = g // gpsub, (g % gpsub) * _L
            dst_v[sub, pl.ds(col, _L)] = dest
            tok_v[sub, pl.ds(col, _L)] = (base + g * _L + iota) >> 1

        pltpu.sync_copy(rix_v, rix_hbm.at[pl.ds(base, ca)])
        pltpu.sync_copy(wk_v, wk_hbm.at[pl.ds(base, ca)])

        bufs = (buf_a, buf_b)
        gsems = (gsem_a, gsem_b)
        gathers = [None] * nsub
        gathers[0] = pltpu.async_copy(x_hbm.at[tok_v.at[0]], bufs[0], gsems[0])
        for sub in range(nsub):
            buf = bufs[sub % 2]
            gathers[sub].wait()
            scat = pltpu.async_copy(buf, xe_hbm.at[dst_v.at[sub]], ssem)
            if sub + 1 < nsub:
                gathers[sub + 1] = pltpu.async_copy(
                    x_hbm.at[tok_v.at[sub + 1]], bufs[(sub + 1) % 2],
                    gsems[(sub + 1) % 2])
            scat.wait()

    return pack


def _mlp_body(x_ref, w1_ref, b1_ref, w2_ref, b2_ref, o_ref, *, ns, fs):
    f = pl.program_id(1)
    x = x_ref[...].astype(jnp.bfloat16)
    contrib = None
    for s in range(ns):
        sl = slice(s * fs, (s + 1) * fs)
        w1 = w1_ref[0][:, sl].astype(jnp.bfloat16)
        h = jax.nn.gelu(jnp.dot(x, w1, preferred_element_type=jnp.float32)
                        + b1_ref[0][:, sl])
        c = jnp.dot(h.astype(jnp.bfloat16), w2_ref[0][sl, :].astype(jnp.bfloat16),
                    preferred_element_type=jnp.float32)
        contrib = c if contrib is None else contrib + c

    @pl.when(f == 0)
    def _():
        o_ref[...] = contrib + b2_ref[0]

    @pl.when(f != 0)
    def _():
        o_ref[...] += contrib


def _mlp(xe, w1, b1, w2, b2, cap):
    e, h, f = w1.shape
    ft = 1024
    nf = f // ft
    return pl.pallas_call(
        functools.partial(_mlp_body, ns=4, fs=ft // 4),
        grid=(e, nf),
        in_specs=[
            pl.BlockSpec((cap, h), lambda i, j: (i, 0)),
            pl.BlockSpec((1, h, ft), lambda i, j: (i, 0, j)),
            pl.BlockSpec((1, 1, ft), lambda i, j: (i, 0, j)),
            pl.BlockSpec((1, ft, h), lambda i, j: (i, j, 0)),
            pl.BlockSpec((1, 1, h), lambda i, j: (i, 0, 0)),
        ],
        out_specs=pl.BlockSpec((cap, h), lambda i, j: (i, 0)),
        out_shape=jax.ShapeDtypeStruct((e * cap + _NW, h), jnp.float32),
        compiler_params=pltpu.CompilerParams(
            dimension_semantics=("parallel", "arbitrary")),
    )(xe, w1, b1.reshape(e, 1, f), w2, b2.reshape(e, 1, h))


def _make_combine(n, h, ecap):
    tpw = n // _NW
    tsc = 8
    nsub = tpw // tsc
    mesh = plsc.VectorSubcoreMesh(core_axis_name="c", subcore_axis_name="s")

    @functools.partial(
        pl.kernel,
        mesh=mesh,
        out_type=jax.ShapeDtypeStruct((n, h), jnp.float32),
        scratch_types=[
            pltpu.VMEM((2 * tpw,), jnp.int32),
            pltpu.VMEM((2 * tpw,), jnp.float32),
            pltpu.VMEM((2 * tsc, h), jnp.float32),
            pltpu.VMEM((2 * tsc, h), jnp.float32),
            pltpu.VMEM((tsc, h), jnp.float32),
            pltpu.VMEM((tsc, h), jnp.float32),
            pltpu.SemaphoreType.DMA,
            pltpu.SemaphoreType.DMA,
            pltpu.SemaphoreType.DMA,
            pltpu.SemaphoreType.DMA,
        ],
        compiler_params=pltpu.CompilerParams(needs_layout_passes=False),
    )
    def combine(y_hbm, rix_hbm, wk_hbm, out_hbm, rix_v, wk_v, ybuf_a, ybuf_b,
                obuf_a, obuf_b, sem_a, sem_b, osem_a, osem_b):
        wid = lax.axis_index("s") * _NC + lax.axis_index("c")
        tbase = wid * tpw
        abase = 2 * tbase
        zeros = jnp.zeros((_L,), jnp.float32)
        for v in range(h // _L):
            obuf_a[0, pl.ds(v * _L, _L)] = zeros
        pltpu.sync_copy(obuf_a.at[pl.ds(0, 1)],
                        y_hbm.at[pl.ds(ecap + wid, 1)])
        pltpu.sync_copy(rix_hbm.at[pl.ds(abase, 2 * tpw)], rix_v)
        pltpu.sync_copy(wk_hbm.at[pl.ds(abase, 2 * tpw)], wk_v)

        bufs = (ybuf_a, ybuf_b)
        sems = (sem_a, sem_b)
        obufs = (obuf_a, obuf_b)
        osems = (osem_a, osem_b)

        def g_idx(sub):
            return rix_v.at[pl.ds(sub * 2 * tsc, 2 * tsc)]

        for b in range(2):
            pltpu.async_copy(y_hbm.at[g_idx(b)], bufs[b], sems[b])

        @pl.loop(0, nsub, step=2)
        def _(outer):
            for b in range(2):
                sub = outer + b
                ybuf, obuf = bufs[b], obufs[b]
                pltpu.make_async_copy(y_hbm.at[g_idx(sub)], ybuf,
                                      sems[b]).wait()

                @pl.when(sub >= 2)
                def _():
                    pltpu.make_async_copy(
                        obuf, out_hbm.at[pl.ds(tbase, tsc)], osems[b]).wait()

                for i in range(tsc):
                    j0 = jnp.broadcast_to(sub * 2 * tsc + 2 * i,
                                          (_L,)).astype(jnp.int32)
                    w0 = plsc.load_gather(wk_v, [j0])
                    w1 = plsc.load_gather(wk_v, [j0 + 1])
                    for v in range(h // _L):
                        ya = ybuf[2 * i, pl.ds(v * _L, _L)]
                        yb = ybuf[2 * i + 1, pl.ds(v * _L, _L)]
                        obuf[i, pl.ds(v * _L, _L)] = ya * w0 + yb * w1

                pltpu.async_copy(obuf,
                                 out_hbm.at[pl.ds(tbase + sub * tsc, tsc)],
                                 osems[b])

                @pl.when(sub + 2 < nsub)
                def _():
                    pltpu.async_copy(y_hbm.at[g_idx(sub + 2)], ybuf, sems[b])

        for b in range(2):
            pltpu.make_async_copy(obufs[b], out_hbm.at[pl.ds(tbase, tsc)],
                                  osems[b]).wait()

    return combine


def kernel(x, Wr, W1, b1, W2, b2):
    bx, tx, hx = x.shape
    n = bx * tx
    e = Wr.shape[1]
    cap = _capacity(n)
    x_flat = x.reshape(n, hx)

    eidx, w, counts = _router(x_flat, Wr)
    pack = _make_pack(n, hx, e, cap)
    xe, rix, wk = pack(eidx.reshape(-1), w.reshape(-1),
                       counts.reshape(-1), x_flat)
    y = _mlp(xe, W1, b1, W2, b2, cap)
    combine = _make_combine(n, hx, e * cap)
    out = combine(y, rix, wk)
    return out.reshape(bx, tx, hx)


# speedup vs baseline: 1.7249x; 1.1870x over previous
import functools
import math

import jax
import jax.numpy as jnp
from jax import lax
from jax.experimental import pallas as pl
from jax.experimental.pallas import tpu as pltpu
from jax.experimental.pallas import tpu_sc as plsc

_NC = 2
_NS = 16
_NW = _NC * _NS
_L = 16

_CF, _RT = 0.25, 128


def _capacity(num_tokens):
    cap = math.ceil(_CF * num_tokens)
    cap = _RT * math.ceil(cap / _RT)
    return max(1, min(cap, num_tokens))


def _router_body(x_ref, wr_ref, eidx_ref, w_ref, cnt_ref, *, tb, e):
    xb = x_ref[...]
    wr = wr_ref[...]
    logits = jnp.dot(xb, wr, preferred_element_type=jnp.float32)
    iota_e = lax.broadcasted_iota(jnp.int32, (tb, e), 1)
    m1 = jnp.max(logits, axis=1, keepdims=True)
    i1 = jnp.min(jnp.where(logits == m1, iota_e, e), axis=1, keepdims=True)
    masked = jnp.where(iota_e == i1, -jnp.inf, logits)
    m2 = jnp.max(masked, axis=1, keepdims=True)
    i2 = jnp.min(jnp.where(masked == m2, iota_e, e), axis=1, keepdims=True)
    w1 = jax.nn.sigmoid(m1 - m2)
    eidx_ref[...] = jnp.concatenate([i1, i2], axis=1)
    w_ref[...] = jnp.concatenate([w1, 1.0 - w1], axis=1)
    oh = (iota_e == i1).astype(jnp.int32) + (iota_e == i2).astype(jnp.int32)
    nch = tb // 128
    for g in range(nch):
        cnt_ref[0, g:g + 1, :] = jnp.sum(
            oh[g * 128:(g + 1) * 128, :], axis=0, keepdims=True)


def _router(x_flat, wr):
    n, h = x_flat.shape
    e = wr.shape[1]
    tb = 512
    grid = (n // tb,)
    return pl.pallas_call(
        functools.partial(_router_body, tb=tb, e=e),
        grid=grid,
        in_specs=[
            pl.BlockSpec((tb, h), lambda i: (i, 0)),
            pl.BlockSpec((h, e), lambda i: (0, 0)),
        ],
        out_specs=[
            pl.BlockSpec((tb, 2), lambda i: (i, 0)),
            pl.BlockSpec((tb, 2), lambda i: (i, 0)),
            pl.BlockSpec((1, tb // 128, e), lambda i: (i, 0, 0)),
        ],
        out_shape=[
            jax.ShapeDtypeStruct((n, 2), jnp.int32),
            jax.ShapeDtypeStruct((n, 2), jnp.float32),
            jax.ShapeDtypeStruct((n // tb, tb // 128, e), jnp.int32),
        ],
    )(x_flat, wr)


def _make_pack(n, h, e, cap):
    a = 2 * n
    ca = a // _NW
    sc = 32
    nsub = ca // sc
    mesh = plsc.VectorSubcoreMesh(core_axis_name="c", subcore_axis_name="s")

    @functools.partial(
        pl.kernel,
        mesh=mesh,
        out_type=[
            jax.ShapeDtypeStruct((e * cap + _NW, h), jnp.float32),
            jax.ShapeDtypeStruct((a,), jnp.int32),
            jax.ShapeDtypeStruct((a,), jnp.float32),
        ],
        scratch_types=[
            pltpu.VMEM((ca,), jnp.int32),
            pltpu.VMEM((ca,), jnp.float32),
            pltpu.VMEM((_NW * e,), jnp.int32),
            pltpu.VMEM((nsub, sc), jnp.int32),
            pltpu.VMEM((nsub, sc), jnp.int32),
            pltpu.VMEM((ca,), jnp.int32),
            pltpu.VMEM((ca,), jnp.float32),
            pltpu.VMEM((sc, h), jnp.float32),
            pltpu.VMEM((sc, h), jnp.float32),
            pltpu.SemaphoreType.DMA,
            pltpu.SemaphoreType.DMA,
            pltpu.SemaphoreType.DMA,
        ],
        compiler_params=pltpu.CompilerParams(needs_layout_passes=False),
    )
    def pack(eidx_hbm, w_hbm, cnt_hbm, x_hbm, xe_hbm, rix_hbm, wk_hbm,
             ev_v, wv_v, ct_v, tok_v, dst_v, rix_v, wk_v, buf_a, buf_b,
             gsem_a, gsem_b, ssem):
        wid = lax.axis_index("s") * _NC + lax.axis_index("c")
        base = wid * ca
        pltpu.sync_copy(eidx_hbm.at[pl.ds(base, ca)], ev_v)
        pltpu.sync_copy(w_hbm.at[pl.ds(base, ca)], wv_v)
        pltpu.sync_copy(cnt_hbm, ct_v)

        iota = lax.iota(jnp.int32, _L)
        carry = []
        for ei in range(e):
            v0 = plsc.load_gather(ct_v, [iota * e + ei])
            v1 = plsc.load_gather(ct_v, [(iota + _L) * e + ei])
            s0 = jnp.sum(jnp.where(iota < wid, v0, 0))
            s1 = jnp.sum(jnp.where(iota + _L < wid, v1, 0))
            carry.append(s0 + s1)

        gpsub = sc // _L
        for g in range(ca // _L):
            ev = ev_v[pl.ds(g * _L, _L)]
            wv = wv_v[pl.ds(g * _L, _L)]
            slot = jnp.zeros((_L,), jnp.int32)
            for ei in range(e):
                ind = ev == ei
                indi = ind.astype(jnp.int32)
                cs = plsc.cumsum(indi)
                slot = slot + jnp.where(ind, cs - indi + carry[ei], 0)
                carry[ei] = carry[ei] + jnp.sum(indi)
            keep = slot < cap
            dest = jnp.where(keep, ev * cap + slot, e * cap + wid)
            rix_v[pl.ds(g * _L, _L)] = dest
            wk_v[pl.ds(g * _L, _L)] = jnp.where(keep, wv, 0.0)
            sub, col = g // gpsub, (g % gpsub) * _L
            dst_v[sub, pl.ds(col, _L)] = dest
            tok_v[sub, pl.ds(col, _L)] = (base + g * _L + iota) >> 1

        pltpu.sync_copy(rix_v, rix_hbm.at[pl.ds(base, ca)])
        pltpu.sync_copy(wk_v, wk_hbm.at[pl.ds(base, ca)])

        bufs = (buf_a, buf_b)
        gsems = (gsem_a, gsem_b)
        gathers = [None] * nsub
        gathers[0] = pltpu.async_copy(x_hbm.at[tok_v.at[0]], bufs[0], gsems[0])
        for sub in range(nsub):
            buf = bufs[sub % 2]
            gathers[sub].wait()
            scat = pltpu.async_copy(buf, xe_hbm.at[dst_v.at[sub]], ssem)
            if sub + 1 < nsub:
                gathers[sub + 1] = pltpu.async_copy(
                    x_hbm.at[tok_v.at[sub + 1]], bufs[(sub + 1) % 2],
                    gsems[(sub + 1) % 2])
            scat.wait()

    return pack


def _mlp_body(x_ref, w1_ref, b1_ref, w2_ref, b2_ref, o_ref, *, ns, fs):
    f = pl.program_id(1)
    x = x_ref[...].astype(jnp.bfloat16)
    contrib = None
    for s in range(ns):
        sl = slice(s * fs, (s + 1) * fs)
        w1 = w1_ref[0][:, sl].astype(jnp.bfloat16)
        h = jax.nn.gelu(jnp.dot(x, w1, preferred_element_type=jnp.float32)
                        + b1_ref[0][:, sl])
        c = jnp.dot(h.astype(jnp.bfloat16), w2_ref[0][sl, :].astype(jnp.bfloat16),
                    preferred_element_type=jnp.float32)
        contrib = c if contrib is None else contrib + c

    @pl.when(f == 0)
    def _():
        o_ref[...] = contrib + b2_ref[0]

    @pl.when(f != 0)
    def _():
        o_ref[...] += contrib


def _mlp(xe, w1, b1, w2, b2, cap):
    e, h, f = w1.shape
    ft = 2048
    nf = f // ft
    return pl.pallas_call(
        functools.partial(_mlp_body, ns=1, fs=ft),
        grid=(e, nf),
        in_specs=[
            pl.BlockSpec((cap, h), lambda i, j: (i, 0)),
            pl.BlockSpec((1, h, ft), lambda i, j: (i, 0, j)),
            pl.BlockSpec((1, 1, ft), lambda i, j: (i, 0, j)),
            pl.BlockSpec((1, ft, h), lambda i, j: (i, j, 0)),
            pl.BlockSpec((1, 1, h), lambda i, j: (i, 0, 0)),
        ],
        out_specs=pl.BlockSpec((cap, h), lambda i, j: (i, 0)),
        out_shape=jax.ShapeDtypeStruct((e * cap + _NW, h), jnp.float32),
        compiler_params=pltpu.CompilerParams(
            dimension_semantics=("parallel", "arbitrary")),
    )(xe, w1, b1.reshape(e, 1, f), w2, b2.reshape(e, 1, h))


def _make_combine(n, h, ecap):
    tpw = n // _NW
    tsc = 8
    nsub = tpw // tsc
    mesh = plsc.VectorSubcoreMesh(core_axis_name="c", subcore_axis_name="s")

    @functools.partial(
        pl.kernel,
        mesh=mesh,
        out_type=jax.ShapeDtypeStruct((n, h), jnp.float32),
        scratch_types=[
            pltpu.VMEM((2 * tpw,), jnp.int32),
            pltpu.VMEM((2 * tpw,), jnp.float32),
            pltpu.VMEM((2 * tsc, h), jnp.float32),
            pltpu.VMEM((2 * tsc, h), jnp.float32),
            pltpu.VMEM((tsc, h), jnp.float32),
            pltpu.VMEM((tsc, h), jnp.float32),
            pltpu.SemaphoreType.DMA,
            pltpu.SemaphoreType.DMA,
            pltpu.SemaphoreType.DMA,
            pltpu.SemaphoreType.DMA,
        ],
        compiler_params=pltpu.CompilerParams(needs_layout_passes=False),
    )
    def combine(y_hbm, rix_hbm, wk_hbm, out_hbm, rix_v, wk_v, ybuf_a, ybuf_b,
                obuf_a, obuf_b, sem_a, sem_b, osem_a, osem_b):
        wid = lax.axis_index("s") * _NC + lax.axis_index("c")
        tbase = wid * tpw
        abase = 2 * tbase
        zeros = jnp.zeros((_L,), jnp.float32)
        for v in range(h // _L):
            obuf_a[0, pl.ds(v * _L, _L)] = zeros
        pltpu.sync_copy(obuf_a.at[pl.ds(0, 1)],
                        y_hbm.at[pl.ds(ecap + wid, 1)])
        pltpu.sync_copy(rix_hbm.at[pl.ds(abase, 2 * tpw)], rix_v)
        pltpu.sync_copy(wk_hbm.at[pl.ds(abase, 2 * tpw)], wk_v)

        bufs = (ybuf_a, ybuf_b)
        sems = (sem_a, sem_b)
        obufs = (obuf_a, obuf_b)
        osems = (osem_a, osem_b)

        def g_idx(sub):
            return rix_v.at[pl.ds(sub * 2 * tsc, 2 * tsc)]

        for b in range(2):
            pltpu.async_copy(y_hbm.at[g_idx(b)], bufs[b], sems[b])

        @pl.loop(0, nsub, step=2)
        def _(outer):
            for b in range(2):
                sub = outer + b
                ybuf, obuf = bufs[b], obufs[b]
                pltpu.make_async_copy(y_hbm.at[g_idx(sub)], ybuf,
                                      sems[b]).wait()

                @pl.when(sub >= 2)
                def _():
                    pltpu.make_async_copy(
                        obuf, out_hbm.at[pl.ds(tbase, tsc)], osems[b]).wait()

                for i in range(tsc):
                    j0 = jnp.broadcast_to(sub * 2 * tsc + 2 * i,
                                          (_L,)).astype(jnp.int32)
                    w0 = plsc.load_gather(wk_v, [j0])
                    w1 = plsc.load_gather(wk_v, [j0 + 1])
                    for v in range(h // _L):
                        ya = ybuf[2 * i, pl.ds(v * _L, _L)]
                        yb = ybuf[2 * i + 1, pl.ds(v * _L, _L)]
                        obuf[i, pl.ds(v * _L, _L)] = ya * w0 + yb * w1

                pltpu.async_copy(obuf,
                                 out_hbm.at[pl.ds(tbase + sub * tsc, tsc)],
                                 osems[b])

                @pl.when(sub + 2 < nsub)
                def _():
                    pltpu.async_copy(y_hbm.at[g_idx(sub + 2)], ybuf, sems[b])

        for b in range(2):
            pltpu.make_async_copy(obufs[b], out_hbm.at[pl.ds(tbase, tsc)],
                                  osems[b]).wait()

    return combine


def kernel(x, Wr, W1, b1, W2, b2):
    bx, tx, hx = x.shape
    n = bx * tx
    e = Wr.shape[1]
    cap = _capacity(n)
    x_flat = x.reshape(n, hx)

    eidx, w, counts = _router(x_flat, Wr)
    pack = _make_pack(n, hx, e, cap)
    xe, rix, wk = pack(eidx.reshape(-1), w.reshape(-1),
                       counts.reshape(-1), x_flat)
    y = _mlp(xe, W1, b1, W2, b2, cap)
    combine = _make_combine(n, hx, e * cap)
    out = combine(y, rix, wk)
    return out.reshape(bx, tx, hx)
